# baseline (speedup 1.0000x reference)
import jax
import jax.numpy as jnp
import numpy as np
from jax.experimental import pallas as pl
from jax.experimental.pallas import tpu as pltpu

N = 10000
E = 320000
D = 128
NG = 64


def _bn_apply(h, p):
    return (h - p["rm"]) / jnp.sqrt(p["rv"] + 1e-5) * p["g"] + p["b"]


def _gcn(h, src, dst, p):
    deg = jnp.zeros((N,), h.dtype).at[dst].add(1.0)
    dis = jnp.where(deg > 0, 1.0 / jnp.sqrt(deg), 0.0)
    norm = dis[src] * dis[dst]
    z = h @ p["W"]
    out = jnp.zeros((N, z.shape[1]), z.dtype).at[dst].add(z[src] * norm[:, None])
    return out + p["b"]


def _gat(h, src, dst, p):
    z = h @ p["W"]
    a = (z * p["as"]).sum(-1)[src] + (z * p["ad"]).sum(-1)[dst]
    a = jax.nn.leaky_relu(a, 0.2)
    m = jax.ops.segment_max(a, dst, num_segments=N)
    ex = jnp.exp(a - m[dst])
    s = jax.ops.segment_sum(ex, dst, num_segments=N)
    alpha = ex / (s[dst] + 1e-16)
    out = jax.ops.segment_sum(z[src] * alpha[:, None], dst, num_segments=N)
    return out + p["b"]


def _bn_in_kernel(h, g, b, rm, rv):
    return (h - rm) / jnp.sqrt(rv + 1e-5) * g + b


def _mlp_head_kernel(pooled_ref, *refs):
    h = pooled_ref[...]
    for i in range(4):
        W, b, g, bb, rm, rv = (r[...] for r in refs[i * 6:(i + 1) * 6])
        h = jax.nn.relu(_bn_in_kernel(jnp.dot(h, W) + b, g, bb, rm, rv))
    wo, bo, out_ref = refs[24], refs[25], refs[26]
    out_ref[...] = jnp.dot(h, wo[...]) + bo[...]


def kernel(x, edge_index, batch, params):
    p = params
    loop = jnp.arange(N, dtype=edge_index.dtype)
    src = jnp.concatenate([edge_index[0], loop])
    dst = jnp.concatenate([edge_index[1], loop])

    l1 = jnp.tanh(_bn_apply(_gcn(x, src, dst, p["gcn_l1"]), p["bn_l1"]))
    l1 = jnp.tanh(_bn_apply(_gat(l1, src, dst, p["gat_l1"]), p["bn_gat_l1"]))
    m1 = jnp.tanh(_bn_apply(_gcn(x, src, dst, p["gcn_m1"]), p["bn_m1"]))
    m2 = jnp.tanh(_bn_apply(_gcn(m1, src, dst, p["gcn_m2"]), p["bn_m2"]))
    m2 = jnp.tanh(_bn_apply(_gat(m2, src, dst, p["gat_m2"]), p["bn_gat_m2"]))
    r1 = jax.nn.relu(_bn_apply(_gcn(x, src, dst, p["gcn_r1"]), p["bn_r1"]))
    r2 = jax.nn.relu(_bn_apply(_gcn(r1, src, dst, p["gcn_r2"]), p["bn_r2"]))
    r3 = jax.nn.relu(_bn_apply(_gcn(r2, src, dst, p["gcn_r3"]), p["bn_r3"]))
    r3 = jax.nn.relu(_bn_apply(_gat(r3, src, dst, p["gat_r3"]), p["bn_gat_r3"]))
    xin = jnp.concatenate([l1, m2, r3], axis=1)
    g = jax.nn.relu(_bn_apply(_gat(xin, src, dst, p["gat_all"]), p["bn_gat_all"]))

    cnt = jax.ops.segment_sum(jnp.ones((N,), g.dtype), batch, num_segments=NG)
    pooled = jax.ops.segment_sum(g, batch, num_segments=NG) / jnp.maximum(cnt, 1.0)[:, None]

    args = [pooled]
    for lin, bn in [(p["lin_gat"], p["bn_lin_gat"]), (p["lin1"], p["bn1"]),
                    (p["lin2"], p["bn2"]), (p["lin3"], p["bn3"])]:
        args += [lin["W"], lin["b"], bn["g"], bn["b"], bn["rm"], bn["rv"]]
    args += [p["out"]["W"], p["out"]["b"]]

    out = pl.pallas_call(
        _mlp_head_kernel,
        out_shape=jax.ShapeDtypeStruct((NG, 1), jnp.float32),
    )(*args)
    return out


# speedup vs baseline: 9.1750x; 9.1750x over previous
import functools

import jax
import jax.numpy as jnp
from jax import lax
from jax.experimental import pallas as pl
from jax.experimental.pallas import tpu as pltpu
from jax.experimental.pallas import tpu_sc as plsc

N = 10000
E = 320000
D = 128
NG = 64

NP = 10240
NC = 2
NS = 16
NW = NC * NS
K = 96
NB = 108
EP = NW * NB * K
BL = 512


def _sc_mesh():
    return plsc.VectorSubcoreMesh(core_axis_name="c", subcore_axis_name="s")


_SC_PARAMS = dict(
    compiler_params=pltpu.CompilerParams(use_tc_tiling_on_sc=False,
                                         needs_layout_passes=False),
)


@functools.lru_cache(maxsize=None)
def _prop_plain(dimp):
    rpt = NP // NS

    def body(z_hbm, srcg_hbm, dstg_hbm, zero_hbm, out_hbm,
             idxs, idxd, rows, acc, sem0, sem1):
        cid = lax.axis_index("c")
        sid = lax.axis_index("s")
        wid = sid * NC + cid
        pltpu.sync_copy(zero_hbm.at[pl.ds(sid * rpt, rpt)],
                        acc.at[pl.ds(sid * rpt, rpt)])
        pltpu.sync_copy(srcg_hbm.at[wid], idxs)
        pltpu.sync_copy(dstg_hbm.at[wid], idxd)
        plsc.subcore_barrier()

        pltpu.async_copy(z_hbm.at[idxs.at[0]], rows.at[0], sem0)

        def step(i, carry):
            j0 = i * 2
            j1 = j0 + 1
            pltpu.make_async_copy(z_hbm.at[idxs.at[j0]], rows.at[0],
                                  sem0).wait()
            pltpu.async_copy(z_hbm.at[idxs.at[j1]], rows.at[1], sem1)
            pltpu.sync_copy(rows.at[0], acc.at[idxd.at[j0]], add=True)
            pltpu.make_async_copy(z_hbm.at[idxs.at[j1]], rows.at[1],
                                  sem1).wait()

            @pl.when(j0 + 2 < NB)
            def _():
                pltpu.async_copy(z_hbm.at[idxs.at[j0 + 2]], rows.at[0], sem0)

            pltpu.sync_copy(rows.at[1], acc.at[idxd.at[j1]], add=True)
            return carry

        lax.fori_loop(0, NB // 2, step, 0)
        plsc.subcore_barrier()
        pltpu.sync_copy(acc.at[pl.ds(sid * rpt, rpt)],
                        out_hbm.at[cid, pl.ds(sid * rpt, rpt)])

    return pl.kernel(
        body,
        out_type=jax.ShapeDtypeStruct((NC, NP, dimp), jnp.float32),
        mesh=_sc_mesh(),
        scratch_types=[
            pltpu.VMEM((NB, K), jnp.int32),
            pltpu.VMEM((NB, K), jnp.int32),
            pltpu.VMEM((2, K, dimp), jnp.float32),
            pltpu.VMEM_SHARED((NP, dimp), jnp.float32),
            pltpu.SemaphoreType.DMA,
            pltpu.SemaphoreType.DMA,
        ],
        **_SC_PARAMS,
    )


@functools.lru_cache(maxsize=None)
def _edge_ex():
    def body(srcg_hbm, dstg_hbm, sa_hbm, da_hbm, c_hbm, ex_hbm,
             idxs, idxd, sa_t, da_t, c_t, exo):
        cid = lax.axis_index("c")
        sid = lax.axis_index("s")
        wid = sid * NC + cid
        pltpu.sync_copy(srcg_hbm.at[wid], idxs)
        pltpu.sync_copy(dstg_hbm.at[wid], idxd)
        pltpu.sync_copy(sa_hbm, sa_t)
        pltpu.sync_copy(da_hbm, da_t)
        pltpu.sync_copy(c_hbm, c_t)

        def step(j, carry):
            def grp(g, c2):
                srcv = idxs[j, pl.ds(g * 16, 16)]
                dstv = idxd[j, pl.ds(g * 16, 16)]
                a = (plsc.load_gather(sa_t, [srcv])
                     + plsc.load_gather(da_t, [dstv]))
                a = jnp.where(a >= 0, a, a * 0.2)
                exo[j, pl.ds(g * 16, 16)] = jnp.exp(
                    a - plsc.load_gather(c_t, [dstv]))
                return c2

            lax.fori_loop(0, K // 16, grp, 0)
            return carry

        lax.fori_loop(0, NB, step, 0)
        pltpu.sync_copy(exo, ex_hbm.at[wid])

    return pl.kernel(
        body,
        out_type=jax.ShapeDtypeStruct((NW, NB, K), jnp.float32),
        mesh=_sc_mesh(),
        scratch_types=[
            pltpu.VMEM((NB, K), jnp.int32),
            pltpu.VMEM((NB, K), jnp.int32),
            pltpu.VMEM((NP,), jnp.float32),
            pltpu.VMEM((NP,), jnp.float32),
            pltpu.VMEM((NP,), jnp.float32),
            pltpu.VMEM((NB, K), jnp.float32),
        ],
        **_SC_PARAMS,
    )


@functools.lru_cache(maxsize=None)
def _prop_weighted(dimp):
    rpt = NP // NS

    def _scale(rows, b, exb):
        dn = lax.GatherDimensionNumbers(
            offset_dims=(), collapsed_slice_dims=(0,), start_index_map=(0,))

        def grp(g, c2):
            ex = exb[b, pl.ds(g * 16, 16)]
            for l in range(16):
                w = lax.gather(ex, jnp.full((16, 1), l, jnp.int32), dn,
                               slice_sizes=(1,),
                               mode=lax.GatherScatterMode.PROMISE_IN_BOUNDS)
                e = g * 16 + l
                for t in range(dimp // 16):
                    sl = pl.ds(t * 16, 16)
                    rows[b, e, sl] = rows[b, e, sl] * w
            return c2

        lax.fori_loop(0, K // 16, grp, 0)

    def body(z_hbm, srcg_hbm, dstg_hbm, ex_hbm, zero_hbm, out_hbm,
             idxd, srcb, exb, rows, acc, semi0, semi1, semg0, semg1):
        cid = lax.axis_index("c")
        sid = lax.axis_index("s")
        wid = sid * NC + cid
        pltpu.sync_copy(zero_hbm.at[pl.ds(sid * rpt, rpt)],
                        acc.at[pl.ds(sid * rpt, rpt)])
        pltpu.sync_copy(dstg_hbm.at[wid], idxd)
        plsc.subcore_barrier()

        def lidx(j, b, sem):
            pltpu.async_copy(srcg_hbm.at[wid, j], srcb.at[b], sem)
            pltpu.async_copy(ex_hbm.at[wid, j], exb.at[b], sem)

        def widx(j, b, sem):
            pltpu.make_async_copy(srcg_hbm.at[wid, j], srcb.at[b], sem).wait()
            pltpu.make_async_copy(ex_hbm.at[wid, j], exb.at[b], sem).wait()

        def gat(j, b, sem):
            pltpu.async_copy(z_hbm.at[srcb.at[b]], rows.at[b], sem)

        def wgat(j, b, sem):
            pltpu.make_async_copy(z_hbm.at[srcb.at[b]], rows.at[b],
                                  sem).wait()

        lidx(0, 0, semi0)
        lidx(1, 1, semi1)
        widx(0, 0, semi0)
        gat(0, 0, semg0)

        def step(i, carry):
            j0 = i * 2
            j1 = j0 + 1
            wgat(j0, 0, semg0)
            widx(j1, 1, semi1)
            gat(j1, 1, semg1)
            _scale(rows, 0, exb)
            pltpu.sync_copy(rows.at[0], acc.at[idxd.at[j0]], add=True)

            @pl.when(j0 + 2 < NB)
            def _():
                lidx(j0 + 2, 0, semi0)

            wgat(j1, 1, semg1)
            _scale(rows, 1, exb)
            pltpu.sync_copy(rows.at[1], acc.at[idxd.at[j1]], add=True)

            @pl.when(j0 + 2 < NB)
            def _():
                widx(j0 + 2, 0, semi0)
                gat(j0 + 2, 0, semg0)
                lidx(j0 + 3, 1, semi1)

            return carry

        lax.fori_loop(0, NB // 2, step, 0)
        plsc.subcore_barrier()
        pltpu.sync_copy(acc.at[pl.ds(sid * rpt, rpt)],
                        out_hbm.at[cid, pl.ds(sid * rpt, rpt)])

    return pl.kernel(
        body,
        out_type=jax.ShapeDtypeStruct((NC, NP, dimp), jnp.float32),
        mesh=_sc_mesh(),
        scratch_types=[
            pltpu.VMEM((NB, K), jnp.int32),
            pltpu.VMEM((2, K), jnp.int32),
            pltpu.VMEM((2, K), jnp.float32),
            pltpu.VMEM((2, K, dimp), jnp.float32),
            pltpu.VMEM_SHARED((NP, dimp), jnp.float32),
            pltpu.SemaphoreType.DMA,
            pltpu.SemaphoreType.DMA,
            pltpu.SemaphoreType.DMA,
            pltpu.SemaphoreType.DMA,
        ],
        **_SC_PARAMS,
    )


def _prop_call(z, srcg, dstg):
    dimp = z.shape[1]
    zero = jnp.zeros((NP, dimp), jnp.float32)
    return _prop_plain(dimp)(z, srcg, dstg, zero)


def _edge_ex_call(srcg, dstg, sa, da, c):
    return _edge_ex()(srcg, dstg, sa, da, c)


def _prop_w_call(zp, srcg, dstg, ex):
    dimp = zp.shape[1]
    zero = jnp.zeros((NP, dimp), jnp.float32)
    return _prop_weighted(dimp)(zp, srcg, dstg, ex, zero)


def _row_specs(*shapes):
    specs = []
    for s in shapes:
        if s is None:
            specs.append(None)
        elif s[0] == "row":
            blk = (BL,) + s[1:]
            specs.append(
                pl.BlockSpec(blk, lambda i, _b=len(s): (i,) + (0,) * (_b - 1)))
        else:
            specs.append(pl.BlockSpec(s, lambda i, _b=len(s): (0,) * _b))
    return specs


def _act(h, act):
    if act == "tanh":
        return jnp.tanh(h)
    return jax.nn.relu(h)


def _bn(h, g, b, rm, rv):
    return (h - rm) / jnp.sqrt(rv + 1e-5) * g + b


def _gcn_pre_call(h, W, dis):
    din, dout = W.shape

    def body(h_ref, w_ref, dis_ref, o_ref):
        o_ref[...] = jnp.dot(h_ref[...], w_ref[...]) * dis_ref[...]

    return pl.pallas_call(
        body,
        grid=(NP // BL,),
        in_specs=_row_specs(("row", din), (din, dout), ("row", 1)),
        out_specs=_row_specs(("row", dout))[0],
        out_shape=jax.ShapeDtypeStruct((NP, dout), jnp.float32),
    )(h, W, dis)


def _gcn_post_call(p0, p1, zs, dis, b, bn, act):
    dout = zs.shape[1]

    def body(p0_ref, p1_ref, zs_ref, dis_ref, b_ref, g_ref, bb_ref, rm_ref,
             rv_ref, o_ref):
        tot = p0_ref[...] + p1_ref[...] + zs_ref[...]
        t = dis_ref[...] * tot + b_ref[...]
        o_ref[...] = _act(_bn(t, g_ref[...], bb_ref[...], rm_ref[...],
                              rv_ref[...]), act)

    return pl.pallas_call(
        body,
        grid=(NP // BL,),
        in_specs=_row_specs(("row", dout), ("row", dout), ("row", dout),
                            ("row", 1), (dout,), (dout,), (dout,), (dout,),
                            (dout,)),
        out_specs=_row_specs(("row", dout))[0],
        out_shape=jax.ShapeDtypeStruct((NP, dout), jnp.float32),
    )(p0, p1, zs, dis, b, bn["g"], bn["b"], bn["rm"], bn["rv"])


def _gat_pre_call(h, W, asv, adv):
    din, dout = W.shape
    dimp = dout + 16

    def body(h_ref, w_ref, as_ref, ad_ref, zp_ref, sa_ref, da_ref):
        z = jnp.dot(h_ref[...], w_ref[...])
        sa_ref[...] = jnp.sum(z * as_ref[...][None, :], axis=1, keepdims=True)
        da_ref[...] = jnp.sum(z * ad_ref[...][None, :], axis=1, keepdims=True)
        onescol = (lax.broadcasted_iota(jnp.int32, (BL, 16), 1) == 0)
        zp_ref[...] = jnp.concatenate([z, onescol.astype(jnp.float32)], axis=1)

    return pl.pallas_call(
        body,
        grid=(NP // BL,),
        in_specs=_row_specs(("row", din), (din, dout), (dout,), (dout,)),
        out_specs=_row_specs(("row", dimp), ("row", 1), ("row", 1)),
        out_shape=(jax.ShapeDtypeStruct((NP, dimp), jnp.float32),
                   jax.ShapeDtypeStruct((NP, 1), jnp.float32),
                   jax.ShapeDtypeStruct((NP, 1), jnp.float32)),
    )(h, W, asv, adv)


def _gat_shift_call(sa, da):
    def body(sa_ref, da_ref, c_ref):
        m = jnp.max(sa_ref[...])
        t = m + da_ref[...]
        c_ref[...] = jnp.where(t >= 0, t, t * 0.2)

    sa2 = sa.reshape(80, 128)
    da2 = da.reshape(80, 128)
    return pl.pallas_call(
        body,
        out_shape=jax.ShapeDtypeStruct((80, 128), jnp.float32),
    )(sa2, da2).reshape(NP, 1)


def _gat_post_call(p0, p1, zp, sa, da, c, b, bn, act):
    dimp = zp.shape[1]
    dout = dimp - 16

    def body(p0_ref, p1_ref, zp_ref, sa_ref, da_ref, c_ref, b_ref, g_ref,
             bb_ref, rm_ref, rv_ref, o_ref):
        aself = sa_ref[...] + da_ref[...]
        aself = jnp.where(aself >= 0, aself, aself * 0.2)
        exs = jnp.exp(aself - c_ref[...])
        tot = p0_ref[...] + p1_ref[...] + exs * zp_ref[...]
        p = tot[:, :dout]
        s = tot[:, dout:dout + 1]
        t = p / (s + 1e-16) + b_ref[...]
        o_ref[...] = _act(_bn(t, g_ref[...], bb_ref[...], rm_ref[...],
                              rv_ref[...]), act)

    return pl.pallas_call(
        body,
        grid=(NP // BL,),
        in_specs=_row_specs(("row", dimp), ("row", dimp), ("row", dimp),
                            ("row", 1), ("row", 1), ("row", 1), (dout,),
                            (dout,), (dout,), (dout,), (dout,)),
        out_specs=_row_specs(("row", dout))[0],
        out_shape=jax.ShapeDtypeStruct((NP, dout), jnp.float32),
    )(p0, p1, zp, sa, da, c, b, bn["g"], bn["b"], bn["rm"], bn["rv"])


def _pool_call(g, batch2d):
    dout = g.shape[1]

    def body(g_ref, bt_ref, pool_ref, cnt_ref):
        i = pl.program_id(0)
        oh = (bt_ref[...] == lax.broadcasted_iota(jnp.int32, (1, NG), 1))
        oh = oh.astype(jnp.float32)

        @pl.when(i == 0)
        def _():
            pool_ref[...] = jnp.zeros_like(pool_ref)
            cnt_ref[...] = jnp.zeros_like(cnt_ref)

        hp = jax.lax.Precision.HIGHEST
        pool_ref[...] += lax.dot_general(oh, g_ref[...],
                                         (((0,), (0,)), ((), ())),
                                         precision=hp)
        ones = jnp.ones((BL, 1), jnp.float32)
        cnt_ref[...] += lax.dot_general(oh, ones, (((0,), (0,)), ((), ())),
                                        precision=hp)

    return pl.pallas_call(
        body,
        grid=(NP // BL,),
        in_specs=_row_specs(("row", dout), ("row", 1)),
        out_specs=_row_specs((NG, dout), (NG, 1)),
        out_shape=(jax.ShapeDtypeStruct((NG, dout), jnp.float32),
                   jax.ShapeDtypeStruct((NG, 1), jnp.float32)),
    )(g, batch2d)


def _head_call(pooled_raw, cnt, p):
    def body(pool_ref, cnt_ref, *refs):
        h = pool_ref[...] / jnp.maximum(cnt_ref[...], 1.0)
        for i in range(4):
            W, b, g, bb, rm, rv = (r[...] for r in refs[i * 6:(i + 1) * 6])
            h = jax.nn.relu(_bn(jnp.dot(h, W) + b, g, bb, rm, rv))
        wo, bo, out_ref = refs[24], refs[25], refs[26]
        out_ref[...] = jnp.dot(h, wo[...]) + bo[...]

    args = [pooled_raw, cnt]
    for lin, bn in [(p["lin_gat"], p["bn_lin_gat"]), (p["lin1"], p["bn1"]),
                    (p["lin2"], p["bn2"]), (p["lin3"], p["bn3"])]:
        args += [lin["W"], lin["b"], bn["g"], bn["b"], bn["rm"], bn["rv"]]
    args += [p["out"]["W"], p["out"]["b"]]
    return pl.pallas_call(
        body,
        out_shape=jax.ShapeDtypeStruct((NG, 1), jnp.float32),
    )(*args)


def _dis_call(d0, d1):
    def body(p0_ref, p1_ref, o_ref):
        deg = 1.0 + p0_ref[...][:, :1] + p1_ref[...][:, :1]
        o_ref[...] = 1.0 / jnp.sqrt(deg)

    return pl.pallas_call(
        body,
        grid=(NP // BL,),
        in_specs=_row_specs(("row", 16), ("row", 16)),
        out_specs=_row_specs(("row", 1))[0],
        out_shape=jax.ShapeDtypeStruct((NP, 1), jnp.float32),
    )(d0, d1)


def kernel(x, edge_index, batch, params):
    p = params
    src = edge_index[0]
    dst = edge_index[1]

    npad = EP - E
    srcp = jnp.concatenate([src, jnp.zeros((npad,), src.dtype)])
    dstp = jnp.concatenate(
        [dst, (N + jnp.arange(npad, dtype=dst.dtype) % (NP - N))])
    srcg = srcp.reshape(NW, NB, K)
    dstg = dstp.reshape(NW, NB, K)

    xp = jnp.pad(x, ((0, NP - N), (0, 0)))
    batch2d = jnp.pad(batch, (0, NP - N), constant_values=NG).reshape(NP, 1)

    degp = _prop_call(jnp.ones((NP, 16), jnp.float32), srcg, dstg)
    dis = _dis_call(degp[0], degp[1])

    def gcn(h, lp, bn, act):
        zs = _gcn_pre_call(h, lp["W"], dis)
        parts = _prop_call(zs, srcg, dstg)
        return _gcn_post_call(parts[0], parts[1], zs, dis, lp["b"], bn, act)

    def gat(h, lp, bn, act):
        zp, sa, da = _gat_pre_call(h, lp["W"], lp["as"], lp["ad"])
        c = _gat_shift_call(sa, da)
        ex = _edge_ex_call(srcg, dstg, sa.reshape(NP), da.reshape(NP),
                           c.reshape(NP))
        parts = _prop_w_call(zp, srcg, dstg, ex)
        return _gat_post_call(parts[0], parts[1], zp, sa, da, c, lp["b"], bn,
                              act)

    l1 = gcn(xp, p["gcn_l1"], p["bn_l1"], "tanh")
    l1 = gat(l1, p["gat_l1"], p["bn_gat_l1"], "tanh")
    m1 = gcn(xp, p["gcn_m1"], p["bn_m1"], "tanh")
    m2 = gcn(m1, p["gcn_m2"], p["bn_m2"], "tanh")
    m2 = gat(m2, p["gat_m2"], p["bn_gat_m2"], "tanh")
    r1 = gcn(xp, p["gcn_r1"], p["bn_r1"], "relu")
    r2 = gcn(r1, p["gcn_r2"], p["bn_r2"], "relu")
    r3 = gcn(r2, p["gcn_r3"], p["bn_r3"], "relu")
    r3 = gat(r3, p["gat_r3"], p["bn_gat_r3"], "relu")
    xin = jnp.concatenate([l1, m2, r3], axis=1)
    g = gat(xin, p["gat_all"], p["bn_gat_all"], "relu")

    pooled_raw, cnt = _pool_call(g, batch2d)
    return _head_call(pooled_raw, cnt, p)


# speedup vs baseline: 11.3485x; 1.2369x over previous
import functools

import jax
import jax.numpy as jnp
from jax import lax
from jax.experimental import pallas as pl
from jax.experimental.pallas import tpu as pltpu
from jax.experimental.pallas import tpu_sc as plsc

N = 10000
E = 320000
D = 128
NG = 64

NP = 10240
NC = 2
NS = 16
NW = NC * NS
K = 96
NB = 108
EP = NW * NB * K
BL = 512


def _sc_mesh():
    return plsc.VectorSubcoreMesh(core_axis_name="c", subcore_axis_name="s")


_SC_PARAMS = dict(
    compiler_params=pltpu.CompilerParams(use_tc_tiling_on_sc=False,
                                         needs_layout_passes=False),
)


@functools.lru_cache(maxsize=None)
def _prop_plain(dimp):
    rpt = NP // NS

    def body(z_hbm, srcg_hbm, dstg_hbm, zero_hbm, out_hbm,
             idxs, idxd, rows, acc, sem0, sem1):
        cid = lax.axis_index("c")
        sid = lax.axis_index("s")
        wid = sid * NC + cid
        pltpu.sync_copy(zero_hbm.at[pl.ds(sid * rpt, rpt)],
                        acc.at[pl.ds(sid * rpt, rpt)])
        pltpu.sync_copy(srcg_hbm.at[wid], idxs)
        pltpu.sync_copy(dstg_hbm.at[wid], idxd)
        plsc.subcore_barrier()

        pltpu.async_copy(z_hbm.at[idxs.at[0]], rows.at[0], sem0)

        def step(i, carry):
            j0 = i * 2
            j1 = j0 + 1
            pltpu.make_async_copy(z_hbm.at[idxs.at[j0]], rows.at[0],
                                  sem0).wait()
            pltpu.async_copy(z_hbm.at[idxs.at[j1]], rows.at[1], sem1)
            pltpu.sync_copy(rows.at[0], acc.at[idxd.at[j0]], add=True)
            pltpu.make_async_copy(z_hbm.at[idxs.at[j1]], rows.at[1],
                                  sem1).wait()

            @pl.when(j0 + 2 < NB)
            def _():
                pltpu.async_copy(z_hbm.at[idxs.at[j0 + 2]], rows.at[0], sem0)

            pltpu.sync_copy(rows.at[1], acc.at[idxd.at[j1]], add=True)
            return carry

        lax.fori_loop(0, NB // 2, step, 0)
        plsc.subcore_barrier()
        pltpu.sync_copy(acc.at[pl.ds(sid * rpt, rpt)],
                        out_hbm.at[cid, pl.ds(sid * rpt, rpt)])

    return pl.kernel(
        body,
        out_type=jax.ShapeDtypeStruct((NC, NP, dimp), jnp.float32),
        mesh=_sc_mesh(),
        scratch_types=[
            pltpu.VMEM((NB, K), jnp.int32),
            pltpu.VMEM((NB, K), jnp.int32),
            pltpu.VMEM((2, K, dimp), jnp.float32),
            pltpu.VMEM_SHARED((NP, dimp), jnp.float32),
            pltpu.SemaphoreType.DMA,
            pltpu.SemaphoreType.DMA,
        ],
        **_SC_PARAMS,
    )


@functools.lru_cache(maxsize=None)
def _edge_ex():
    def body(srcg_hbm, dstg_hbm, sa_hbm, da_hbm, c_hbm, ex_hbm,
             idxs, idxd, sa_t, da_t, c_t, exo):
        cid = lax.axis_index("c")
        sid = lax.axis_index("s")
        wid = sid * NC + cid
        pltpu.sync_copy(srcg_hbm.at[wid], idxs)
        pltpu.sync_copy(dstg_hbm.at[wid], idxd)
        pltpu.sync_copy(sa_hbm, sa_t)
        pltpu.sync_copy(da_hbm, da_t)
        pltpu.sync_copy(c_hbm, c_t)

        def step(j, carry):
            def grp(g, c2):
                srcv = idxs[j, pl.ds(g * 16, 16)]
                dstv = idxd[j, pl.ds(g * 16, 16)]
                a = (plsc.load_gather(sa_t, [srcv])
                     + plsc.load_gather(da_t, [dstv]))
                a = jnp.where(a >= 0, a, a * 0.2)
                exo[j, pl.ds(g * 16, 16)] = jnp.exp(
                    a - plsc.load_gather(c_t, [dstv]))
                return c2

            lax.fori_loop(0, K // 16, grp, 0)
            return carry

        lax.fori_loop(0, NB, step, 0)
        pltpu.sync_copy(exo, ex_hbm.at[wid])

    return pl.kernel(
        body,
        out_type=jax.ShapeDtypeStruct((NW, NB, K), jnp.float32),
        mesh=_sc_mesh(),
        scratch_types=[
            pltpu.VMEM((NB, K), jnp.int32),
            pltpu.VMEM((NB, K), jnp.int32),
            pltpu.VMEM((NP,), jnp.float32),
            pltpu.VMEM((NP,), jnp.float32),
            pltpu.VMEM((NP,), jnp.float32),
            pltpu.VMEM((NB, K), jnp.float32),
        ],
        **_SC_PARAMS,
    )


@functools.lru_cache(maxsize=None)
def _prop_weighted(dimp):
    rpt = NP // NS

    def _scale(rows, b, exb):
        dn = lax.GatherDimensionNumbers(
            offset_dims=(), collapsed_slice_dims=(0,), start_index_map=(0,))

        def grp(g, c2):
            ex = exb[b, pl.ds(g * 16, 16)]
            for l in range(16):
                w = lax.gather(ex, jnp.full((16, 1), l, jnp.int32), dn,
                               slice_sizes=(1,),
                               mode=lax.GatherScatterMode.PROMISE_IN_BOUNDS)
                e = g * 16 + l
                for t in range(dimp // 16):
                    sl = pl.ds(t * 16, 16)
                    rows[b, e, sl] = rows[b, e, sl] * w
            return c2

        lax.fori_loop(0, K // 16, grp, 0)

    def body(z_hbm, srcg_hbm, dstg_hbm, ex_hbm, zero_hbm, out_hbm,
             idxd, srcb, exb, rows, acc, semi0, semi1, semg0, semg1):
        cid = lax.axis_index("c")
        sid = lax.axis_index("s")
        wid = sid * NC + cid
        pltpu.sync_copy(zero_hbm.at[pl.ds(sid * rpt, rpt)],
                        acc.at[pl.ds(sid * rpt, rpt)])
        pltpu.sync_copy(dstg_hbm.at[wid], idxd)
        plsc.subcore_barrier()

        def lidx(j, b, sem):
            pltpu.async_copy(srcg_hbm.at[wid, j], srcb.at[b], sem)
            pltpu.async_copy(ex_hbm.at[wid, j], exb.at[b], sem)

        def widx(j, b, sem):
            pltpu.make_async_copy(srcg_hbm.at[wid, j], srcb.at[b], sem).wait()
            pltpu.make_async_copy(ex_hbm.at[wid, j], exb.at[b], sem).wait()

        def gat(j, b, sem):
            pltpu.async_copy(z_hbm.at[srcb.at[b]], rows.at[b], sem)

        def wgat(j, b, sem):
            pltpu.make_async_copy(z_hbm.at[srcb.at[b]], rows.at[b],
                                  sem).wait()

        lidx(0, 0, semi0)
        lidx(1, 1, semi1)
        widx(0, 0, semi0)
        gat(0, 0, semg0)

        def step(i, carry):
            j0 = i * 2
            j1 = j0 + 1
            wgat(j0, 0, semg0)
            widx(j1, 1, semi1)
            gat(j1, 1, semg1)
            _scale(rows, 0, exb)
            pltpu.sync_copy(rows.at[0], acc.at[idxd.at[j0]], add=True)

            @pl.when(j0 + 2 < NB)
            def _():
                lidx(j0 + 2, 0, semi0)

            wgat(j1, 1, semg1)
            _scale(rows, 1, exb)
            pltpu.sync_copy(rows.at[1], acc.at[idxd.at[j1]], add=True)

            @pl.when(j0 + 2 < NB)
            def _():
                widx(j0 + 2, 0, semi0)
                gat(j0 + 2, 0, semg0)
                lidx(j0 + 3, 1, semi1)

            return carry

        lax.fori_loop(0, NB // 2, step, 0)
        plsc.subcore_barrier()
        pltpu.sync_copy(acc.at[pl.ds(sid * rpt, rpt)],
                        out_hbm.at[cid, pl.ds(sid * rpt, rpt)])

    return pl.kernel(
        body,
        out_type=jax.ShapeDtypeStruct((NC, NP, dimp), jnp.float32),
        mesh=_sc_mesh(),
        scratch_types=[
            pltpu.VMEM((NB, K), jnp.int32),
            pltpu.VMEM((2, K), jnp.int32),
            pltpu.VMEM((2, K), jnp.float32),
            pltpu.VMEM((2, K, dimp), jnp.float32),
            pltpu.VMEM_SHARED((NP, dimp), jnp.float32),
            pltpu.SemaphoreType.DMA,
            pltpu.SemaphoreType.DMA,
            pltpu.SemaphoreType.DMA,
            pltpu.SemaphoreType.DMA,
        ],
        **_SC_PARAMS,
    )


def _prop_call(z, srcg, dstg):
    dimp = z.shape[1]
    zero = jnp.zeros((NP, dimp), jnp.float32)
    return _prop_plain(dimp)(z, srcg, dstg, zero)


def _edge_ex_call(srcg, dstg, sa, da, c):
    return _edge_ex()(srcg, dstg, sa, da, c)


def _prop_w_call(zp, srcg, dstg, ex):
    dimp = zp.shape[1]
    zero = jnp.zeros((NP, dimp), jnp.float32)
    return _prop_weighted(dimp)(zp, srcg, dstg, ex, zero)


def _row_specs(*shapes):
    specs = []
    for s in shapes:
        if s is None:
            specs.append(None)
        elif s[0] == "row":
            blk = (BL,) + s[1:]
            specs.append(
                pl.BlockSpec(blk, lambda i, _b=len(s): (i,) + (0,) * (_b - 1)))
        else:
            specs.append(pl.BlockSpec(s, lambda i, _b=len(s): (0,) * _b))
    return specs


def _act(h, act):
    if act == "tanh":
        return jnp.tanh(h)
    return jax.nn.relu(h)


def _bn(h, g, b, rm, rv):
    return (h - rm) / jnp.sqrt(rv + 1e-5) * g + b


def _scale_call(h, dis):
    dout = h.shape[1]

    def body(h_ref, dis_ref, o_ref):
        o_ref[...] = h_ref[...] * dis_ref[...]

    return pl.pallas_call(
        body,
        grid=(NP // BL,),
        in_specs=_row_specs(("row", dout), ("row", 1)),
        out_specs=_row_specs(("row", dout))[0],
        out_shape=jax.ShapeDtypeStruct((NP, dout), jnp.float32),
    )(h, dis)


def _gcn_fused_post_call(p0, p1, xs, dis, W, b, bn, act):
    din, dout = W.shape

    def body(p0_ref, p1_ref, xs_ref, dis_ref, w_ref, b_ref, g_ref, bb_ref,
             rm_ref, rv_ref, o_ref):
        tot = p0_ref[...] + p1_ref[...] + xs_ref[...]
        t = dis_ref[...] * jnp.dot(tot, w_ref[...]) + b_ref[...]
        o_ref[...] = _act(_bn(t, g_ref[...], bb_ref[...], rm_ref[...],
                              rv_ref[...]), act)

    return pl.pallas_call(
        body,
        grid=(NP // BL,),
        in_specs=_row_specs(("row", din), ("row", din), ("row", din),
                            ("row", 1), (din, dout), (dout,), (dout,),
                            (dout,), (dout,), (dout,)),
        out_specs=_row_specs(("row", dout))[0],
        out_shape=jax.ShapeDtypeStruct((NP, dout), jnp.float32),
    )(p0, p1, xs, dis, W, b, bn["g"], bn["b"], bn["rm"], bn["rv"])


def _gcn_pre_call(h, W, dis):
    din, dout = W.shape

    def body(h_ref, w_ref, dis_ref, o_ref):
        o_ref[...] = jnp.dot(h_ref[...], w_ref[...]) * dis_ref[...]

    return pl.pallas_call(
        body,
        grid=(NP // BL,),
        in_specs=_row_specs(("row", din), (din, dout), ("row", 1)),
        out_specs=_row_specs(("row", dout))[0],
        out_shape=jax.ShapeDtypeStruct((NP, dout), jnp.float32),
    )(h, W, dis)


def _gcn_post_call(p0, p1, zs, dis, b, bn, act):
    dout = zs.shape[1]

    def body(p0_ref, p1_ref, zs_ref, dis_ref, b_ref, g_ref, bb_ref, rm_ref,
             rv_ref, o_ref):
        tot = p0_ref[...] + p1_ref[...] + zs_ref[...]
        t = dis_ref[...] * tot + b_ref[...]
        o_ref[...] = _act(_bn(t, g_ref[...], bb_ref[...], rm_ref[...],
                              rv_ref[...]), act)

    return pl.pallas_call(
        body,
        grid=(NP // BL,),
        in_specs=_row_specs(("row", dout), ("row", dout), ("row", dout),
                            ("row", 1), (dout,), (dout,), (dout,), (dout,),
                            (dout,)),
        out_specs=_row_specs(("row", dout))[0],
        out_shape=jax.ShapeDtypeStruct((NP, dout), jnp.float32),
    )(p0, p1, zs, dis, b, bn["g"], bn["b"], bn["rm"], bn["rv"])


def _gat_pre_call(h, W, asv, adv):
    din, dout = W.shape
    dimp = dout + 16

    def body(h_ref, w_ref, as_ref, ad_ref, zp_ref, sa_ref, da_ref):
        z = jnp.dot(h_ref[...], w_ref[...])
        sa_ref[...] = jnp.sum(z * as_ref[...][None, :], axis=1, keepdims=True)
        da_ref[...] = jnp.sum(z * ad_ref[...][None, :], axis=1, keepdims=True)
        onescol = (lax.broadcasted_iota(jnp.int32, (BL, 16), 1) == 0)
        zp_ref[...] = jnp.concatenate([z, onescol.astype(jnp.float32)], axis=1)

    return pl.pallas_call(
        body,
        grid=(NP // BL,),
        in_specs=_row_specs(("row", din), (din, dout), (dout,), (dout,)),
        out_specs=_row_specs(("row", dimp), ("row", 1), ("row", 1)),
        out_shape=(jax.ShapeDtypeStruct((NP, dimp), jnp.float32),
                   jax.ShapeDtypeStruct((NP, 1), jnp.float32),
                   jax.ShapeDtypeStruct((NP, 1), jnp.float32)),
    )(h, W, asv, adv)


def _gat_shift_call(sa, da):
    def body(sa_ref, da_ref, c_ref):
        m = jnp.max(sa_ref[...])
        t = m + da_ref[...]
        c_ref[...] = jnp.where(t >= 0, t, t * 0.2)

    sa2 = sa.reshape(80, 128)
    da2 = da.reshape(80, 128)
    return pl.pallas_call(
        body,
        out_shape=jax.ShapeDtypeStruct((80, 128), jnp.float32),
    )(sa2, da2).reshape(NP, 1)


def _gat_post_call(p0, p1, zp, sa, da, c, b, bn, act):
    dimp = zp.shape[1]
    dout = dimp - 16

    def body(p0_ref, p1_ref, zp_ref, sa_ref, da_ref, c_ref, b_ref, g_ref,
             bb_ref, rm_ref, rv_ref, o_ref):
        aself = sa_ref[...] + da_ref[...]
        aself = jnp.where(aself >= 0, aself, aself * 0.2)
        exs = jnp.exp(aself - c_ref[...])
        tot = p0_ref[...] + p1_ref[...] + exs * zp_ref[...]
        p = tot[:, :dout]
        s = tot[:, dout:dout + 1]
        t = p / (s + 1e-16) + b_ref[...]
        o_ref[...] = _act(_bn(t, g_ref[...], bb_ref[...], rm_ref[...],
                              rv_ref[...]), act)

    return pl.pallas_call(
        body,
        grid=(NP // BL,),
        in_specs=_row_specs(("row", dimp), ("row", dimp), ("row", dimp),
                            ("row", 1), ("row", 1), ("row", 1), (dout,),
                            (dout,), (dout,), (dout,), (dout,)),
        out_specs=_row_specs(("row", dout))[0],
        out_shape=jax.ShapeDtypeStruct((NP, dout), jnp.float32),
    )(p0, p1, zp, sa, da, c, b, bn["g"], bn["b"], bn["rm"], bn["rv"])


def _pool_call(g, batch2d):
    dout = g.shape[1]

    def body(g_ref, bt_ref, pool_ref, cnt_ref):
        i = pl.program_id(0)
        oh = (bt_ref[...] == lax.broadcasted_iota(jnp.int32, (1, NG), 1))
        oh = oh.astype(jnp.float32)

        @pl.when(i == 0)
        def _():
            pool_ref[...] = jnp.zeros_like(pool_ref)
            cnt_ref[...] = jnp.zeros_like(cnt_ref)

        hp = jax.lax.Precision.HIGHEST
        pool_ref[...] += lax.dot_general(oh, g_ref[...],
                                         (((0,), (0,)), ((), ())),
                                         precision=hp)
        ones = jnp.ones((BL, 1), jnp.float32)
        cnt_ref[...] += lax.dot_general(oh, ones, (((0,), (0,)), ((), ())),
                                        precision=hp)

    return pl.pallas_call(
        body,
        grid=(NP // BL,),
        in_specs=_row_specs(("row", dout), ("row", 1)),
        out_specs=_row_specs((NG, dout), (NG, 1)),
        out_shape=(jax.ShapeDtypeStruct((NG, dout), jnp.float32),
                   jax.ShapeDtypeStruct((NG, 1), jnp.float32)),
    )(g, batch2d)


def _head_call(pooled_raw, cnt, p):
    def body(pool_ref, cnt_ref, *refs):
        h = pool_ref[...] / jnp.maximum(cnt_ref[...], 1.0)
        for i in range(4):
            W, b, g, bb, rm, rv = (r[...] for r in refs[i * 6:(i + 1) * 6])
            h = jax.nn.relu(_bn(jnp.dot(h, W) + b, g, bb, rm, rv))
        wo, bo, out_ref = refs[24], refs[25], refs[26]
        out_ref[...] = jnp.dot(h, wo[...]) + bo[...]

    args = [pooled_raw, cnt]
    for lin, bn in [(p["lin_gat"], p["bn_lin_gat"]), (p["lin1"], p["bn1"]),
                    (p["lin2"], p["bn2"]), (p["lin3"], p["bn3"])]:
        args += [lin["W"], lin["b"], bn["g"], bn["b"], bn["rm"], bn["rv"]]
    args += [p["out"]["W"], p["out"]["b"]]
    return pl.pallas_call(
        body,
        out_shape=jax.ShapeDtypeStruct((NG, 1), jnp.float32),
    )(*args)


def _dis_call(d0, d1):
    def body(p0_ref, p1_ref, o_ref):
        deg = 1.0 + p0_ref[...][:, :1] + p1_ref[...][:, :1]
        o_ref[...] = 1.0 / jnp.sqrt(deg)

    return pl.pallas_call(
        body,
        grid=(NP // BL,),
        in_specs=_row_specs(("row", 16), ("row", 16)),
        out_specs=_row_specs(("row", 1))[0],
        out_shape=jax.ShapeDtypeStruct((NP, 1), jnp.float32),
    )(d0, d1)


def kernel(x, edge_index, batch, params):
    p = params
    src = edge_index[0]
    dst = edge_index[1]

    npad = EP - E
    srcp = jnp.concatenate([src, jnp.zeros((npad,), src.dtype)])
    dstp = jnp.concatenate(
        [dst, (N + jnp.arange(npad, dtype=dst.dtype) % (NP - N))])
    srcg = srcp.reshape(NW, NB, K)
    dstg = dstp.reshape(NW, NB, K)

    xp = jnp.pad(x, ((0, NP - N), (0, 0)))
    batch2d = jnp.pad(batch, (0, NP - N), constant_values=NG).reshape(NP, 1)

    degp = _prop_call(jnp.ones((NP, 16), jnp.float32), srcg, dstg)
    dis = _dis_call(degp[0], degp[1])

    def gcn(h, lp, bn, act):
        zs = _gcn_pre_call(h, lp["W"], dis)
        parts = _prop_call(zs, srcg, dstg)
        return _gcn_post_call(parts[0], parts[1], zs, dis, lp["b"], bn, act)

    xs = _scale_call(xp, dis)
    px = _prop_call(xs, srcg, dstg)

    def gcn_from_x(lp, bn, act):
        return _gcn_fused_post_call(px[0], px[1], xs, dis, lp["W"], lp["b"],
                                    bn, act)

    def gat(h, lp, bn, act):
        zp, sa, da = _gat_pre_call(h, lp["W"], lp["as"], lp["ad"])
        c = _gat_shift_call(sa, da)
        ex = _edge_ex_call(srcg, dstg, sa.reshape(NP), da.reshape(NP),
                           c.reshape(NP))
        parts = _prop_w_call(zp, srcg, dstg, ex)
        return _gat_post_call(parts[0], parts[1], zp, sa, da, c, lp["b"], bn,
                              act)

    l1 = gcn_from_x(p["gcn_l1"], p["bn_l1"], "tanh")
    l1 = gat(l1, p["gat_l1"], p["bn_gat_l1"], "tanh")
    m1 = gcn_from_x(p["gcn_m1"], p["bn_m1"], "tanh")
    m2 = gcn(m1, p["gcn_m2"], p["bn_m2"], "tanh")
    m2 = gat(m2, p["gat_m2"], p["bn_gat_m2"], "tanh")
    r1 = gcn_from_x(p["gcn_r1"], p["bn_r1"], "relu")
    r2 = gcn(r1, p["gcn_r2"], p["bn_r2"], "relu")
    r3 = gcn(r2, p["gcn_r3"], p["bn_r3"], "relu")
    r3 = gat(r3, p["gat_r3"], p["bn_gat_r3"], "relu")
    xin = jnp.concatenate([l1, m2, r3], axis=1)
    g = gat(xin, p["gat_all"], p["bn_gat_all"], "relu")

    pooled_raw, cnt = _pool_call(g, batch2d)
    return _head_call(pooled_raw, cnt, p)
